# baseline (speedup 1.0000x reference)
import functools

import jax
import jax.numpy as jnp
from jax import lax
from jax.experimental import pallas as pl
from jax.experimental.pallas import tpu as pltpu
from jax.experimental.pallas import tpu_sc as plsc

_CHUNK = 128
_LANES = 16


@functools.lru_cache(maxsize=None)
def _make_gather(V, D, B):
    info = plsc.get_sparse_core_info()
    NC, NS = info.num_cores, info.num_subcores
    NW = NC * NS
    b_per_w = B // NW
    n_chunks = b_per_w // _CHUNK
    slab_w = 8 * D
    mesh = plsc.VectorSubcoreMesh(core_axis_name="c", subcore_axis_name="s")

    @functools.partial(
        pl.kernel,
        mesh=mesh,
        out_type=jax.ShapeDtypeStruct((D, B), jnp.float32),
        compiler_params=pltpu.CompilerParams(needs_layout_passes=False),
        scratch_types=[
            pltpu.VMEM((b_per_w,), jnp.int32),
            pltpu.VMEM((b_per_w,), jnp.int32),
            pltpu.VMEM((_CHUNK, slab_w), jnp.float32),
            pltpu.VMEM((_CHUNK, slab_w), jnp.float32),
            pltpu.VMEM((D, b_per_w), jnp.float32),
            pltpu.SemaphoreType.DMA,
            pltpu.SemaphoreType.DMA,
        ],
    )
    def gather_kernel(tbl_hbm, idx_hbm, out_hbm, idx_v, t_v, slab0, slab1,
                      outT_v, sem0, sem1):
        wid = lax.axis_index("s") * NC + lax.axis_index("c")
        base = wid * b_per_w
        pltpu.sync_copy(idx_hbm.at[pl.ds(base, b_per_w)], idx_v)

        def slab_ids(g, _):
            off = g * _LANES
            v = idx_v[pl.ds(off, _LANES)]
            t_v[pl.ds(off, _LANES)] = lax.shift_right_logical(v, 3)
            return _

        lax.fori_loop(0, b_per_w // _LANES, slab_ids, None)

        slabs = (slab0, slab1)
        sems = (sem0, sem1)

        def fire(j):
            return pltpu.async_copy(
                tbl_hbm.at[t_v.at[pl.ds(j * _CHUNK, _CHUNK)]],
                slabs[j % 2],
                sems[j % 2],
            )

        lane = lax.broadcasted_iota(jnp.int32, (_LANES,), 0)
        copies = {0: fire(0)}
        for j in range(n_chunks):
            if j + 1 < n_chunks:
                copies[j + 1] = fire(j + 1)
            copies.pop(j).wait()
            slab = slabs[j % 2]

            def extract(g, _, j=j, slab=slab):
                off = g * _LANES
                s_vec = lax.bitwise_and(idx_v[pl.ds(off, _LANES)], 7)
                pos0 = lax.shift_left(s_vec, 5)
                i_vec = lane + g * _LANES - (j * _CHUNK // _LANES) * _LANES
                for c in range(D):
                    vals = plsc.load_gather(slab, [i_vec, pos0 + c])
                    outT_v[c, pl.ds(off, _LANES)] = vals
                return _

            lax.fori_loop(j * (_CHUNK // _LANES), (j + 1) * (_CHUNK // _LANES),
                          extract, None)
        pltpu.sync_copy(outT_v, out_hbm.at[:, pl.ds(base, b_per_w)])

    return gather_kernel


def kernel(labels, table):
    B = labels.shape[0]
    V, D = table.shape
    tbl2 = table.reshape(V // 8, 8 * D)
    idx = labels.astype(jnp.int32)
    out_t = _make_gather(V, D, B)(tbl2, idx)
    return out_t.T[:, None, :]
